# Initial kernel scaffold; baseline (speedup 1.0000x reference)
import jax
import jax.numpy as jnp
from jax.experimental import pallas as pl


def kernel(features, x, Adj_param, W1, b1, W2, b2):
    raise NotImplementedError("write your pallas kernel here")


# speedup vs baseline: 1.3389x; 1.3389x over previous
import jax
import jax.numpy as jnp
from jax.experimental import pallas as pl

EOS = 1e-10
BM = 512
BN = 512


def _elu1(a):
    return jnp.where(a > 0, a + 1.0, jnp.exp(a))


def _stats_kernel(a_ref, rowsum_ref, colpart_ref):
    j = pl.program_id(1)
    e = _elu1(a_ref[:])
    rs = jnp.sum(e, axis=1, keepdims=True)

    @pl.when(j == 0)
    def _():
        rowsum_ref[:] = rs

    @pl.when(j != 0)
    def _():
        rowsum_ref[:] += rs

    colpart_ref[:] = jnp.sum(e, axis=0).reshape(1, 1, -1)


def _lin1_kernel(x_ref, w_ref, b_ref, o_ref):
    o_ref[:] = jnp.dot(x_ref[:], w_ref[:], preferred_element_type=jnp.float32) + b_ref[:]


def _lin2_kernel(y_ref, w_ref, b_ref, o_ref):
    h = jnp.maximum(y_ref[:], 0.0)
    o_ref[:] = jnp.dot(h, w_ref[:], preferred_element_type=jnp.float32) + b_ref[:]


def _main_kernel(a_ref, at_ref, h1_ref, dc_ref, dr_ref, adjn_ref, y1_ref):
    j = pl.program_id(1)
    e = 0.5 * (_elu1(a_ref[:]) + _elu1(at_ref[:]).T)
    adjn = dc_ref[:] * e * dr_ref[:]
    adjn_ref[:] = adjn
    c = jnp.dot(adjn, h1_ref[:], preferred_element_type=jnp.float32)

    @pl.when(j == 0)
    def _():
        y1_ref[:] = c

    @pl.when(j != 0)
    def _():
        y1_ref[:] += c


def _mm_kernel(adjn_ref, h2_ref, o_ref):
    j = pl.program_id(1)
    c = jnp.dot(adjn_ref[:], h2_ref[:], preferred_element_type=jnp.float32)

    @pl.when(j == 0)
    def _():
        o_ref[:] = c

    @pl.when(j != 0)
    def _():
        o_ref[:] += c


def kernel(features, x, Adj_param, W1, b1, W2, b2):
    N = Adj_param.shape[0]
    in_dim = x.shape[1]
    hid = W1.shape[1]
    ncls = W2.shape[1]
    nb = N // BM

    rowsum, colpart = pl.pallas_call(
        _stats_kernel,
        grid=(nb, nb),
        in_specs=[pl.BlockSpec((BM, BN), lambda i, j: (i, j))],
        out_specs=[
            pl.BlockSpec((BM, 1), lambda i, j: (i, 0)),
            pl.BlockSpec((1, 1, BN), lambda i, j: (i, 0, j)),
        ],
        out_shape=[
            jax.ShapeDtypeStruct((N, 1), jnp.float32),
            jax.ShapeDtypeStruct((nb, 1, N), jnp.float32),
        ],
    )(Adj_param)

    deg = 0.5 * (rowsum[:, 0] + jnp.sum(colpart, axis=(0, 1)))
    dinv = 1.0 / (jnp.sqrt(deg) + EOS)
    dc = dinv[:, None]
    dr = dinv[None, :]

    h1 = pl.pallas_call(
        _lin1_kernel,
        out_shape=jax.ShapeDtypeStruct((N, hid), jnp.float32),
    )(x, W1, b1.reshape(1, hid))

    adjn, y1 = pl.pallas_call(
        _main_kernel,
        grid=(nb, nb),
        in_specs=[
            pl.BlockSpec((BM, BN), lambda i, j: (i, j)),
            pl.BlockSpec((BM, BN), lambda i, j: (j, i)),
            pl.BlockSpec((BN, hid), lambda i, j: (j, 0)),
            pl.BlockSpec((BM, 1), lambda i, j: (i, 0)),
            pl.BlockSpec((1, BN), lambda i, j: (0, j)),
        ],
        out_specs=[
            pl.BlockSpec((BM, BN), lambda i, j: (i, j)),
            pl.BlockSpec((BM, hid), lambda i, j: (i, 0)),
        ],
        out_shape=[
            jax.ShapeDtypeStruct((N, N), jnp.float32),
            jax.ShapeDtypeStruct((N, hid), jnp.float32),
        ],
    )(Adj_param, Adj_param, h1, dc, dr)

    h2 = pl.pallas_call(
        _lin2_kernel,
        out_shape=jax.ShapeDtypeStruct((N, ncls), jnp.float32),
    )(y1, W2, b2.reshape(1, ncls))

    out = pl.pallas_call(
        _mm_kernel,
        grid=(nb, nb),
        in_specs=[
            pl.BlockSpec((BM, BN), lambda i, j: (i, j)),
            pl.BlockSpec((BN, ncls), lambda i, j: (j, 0)),
        ],
        out_specs=pl.BlockSpec((BM, ncls), lambda i, j: (i, 0)),
        out_shape=jax.ShapeDtypeStruct((N, ncls), jnp.float32),
    )(adjn, h2)

    return (out, adjn)


# speedup vs baseline: 2.2190x; 1.6573x over previous
import jax
import jax.numpy as jnp
from jax.experimental import pallas as pl
from jax.experimental.pallas import tpu as pltpu

EOS = 1e-10
BM = 1024
BN = 1024


def _elu1(a):
    return jnp.where(a > 0, a + 1.0, jnp.exp(a))


def _stats_kernel(a_ref, x_ref, w1_ref, b1_ref, rowsum_ref, colpart_ref, h1_ref):
    j = pl.program_id(1)
    e = _elu1(a_ref[:])
    rs = jnp.sum(e, axis=1, keepdims=True)

    @pl.when(j == 0)
    def _():
        rowsum_ref[:] = rs
        h1_ref[:] = (
            jnp.dot(x_ref[:], w1_ref[:], preferred_element_type=jnp.float32)
            + b1_ref[:]
        )

    @pl.when(j != 0)
    def _():
        rowsum_ref[:] += rs

    colpart_ref[:] = jnp.sum(e, axis=0).reshape(1, 1, -1)


def _main_kernel(a_ref, at_ref, h1_ref, dc_ref, dr_ref, adjn_ref, y1_ref):
    j = pl.program_id(1)
    e = 0.5 * (_elu1(a_ref[:]) + _elu1(at_ref[:]).T)
    adjn = dc_ref[:] * e * dr_ref[:]
    adjn_ref[:] = adjn
    c = jnp.dot(adjn, h1_ref[:], preferred_element_type=jnp.float32)

    @pl.when(j == 0)
    def _():
        y1_ref[:] = c

    @pl.when(j != 0)
    def _():
        y1_ref[:] += c


def _out_kernel(adjn_ref, y1_ref, w2_ref, b2_ref, o_ref, h2_ref):
    i = pl.program_id(0)
    j = pl.program_id(1)

    @pl.when(i == 0)
    def _():
        h = jnp.maximum(y1_ref[:], 0.0)
        h2_ref[pl.ds(j * BN, BN), :] = (
            jnp.dot(h, w2_ref[:], preferred_element_type=jnp.float32) + b2_ref[:]
        )

    c = jnp.dot(
        adjn_ref[:], h2_ref[pl.ds(j * BN, BN), :], preferred_element_type=jnp.float32
    )

    @pl.when(j == 0)
    def _():
        o_ref[:] = c

    @pl.when(j != 0)
    def _():
        o_ref[:] += c


def kernel(features, x, Adj_param, W1, b1, W2, b2):
    N = Adj_param.shape[0]
    in_dim = x.shape[1]
    hid = W1.shape[1]
    ncls = W2.shape[1]
    nb = N // BM

    rowsum, colpart, h1 = pl.pallas_call(
        _stats_kernel,
        grid=(nb, nb),
        in_specs=[
            pl.BlockSpec((BM, BN), lambda i, j: (i, j)),
            pl.BlockSpec((BM, in_dim), lambda i, j: (i, 0)),
            pl.BlockSpec((in_dim, hid), lambda i, j: (0, 0)),
            pl.BlockSpec((1, hid), lambda i, j: (0, 0)),
        ],
        out_specs=[
            pl.BlockSpec((BM, 1), lambda i, j: (i, 0)),
            pl.BlockSpec((1, 1, BN), lambda i, j: (i, 0, j)),
            pl.BlockSpec((BM, hid), lambda i, j: (i, 0)),
        ],
        out_shape=[
            jax.ShapeDtypeStruct((N, 1), jnp.float32),
            jax.ShapeDtypeStruct((nb, 1, N), jnp.float32),
            jax.ShapeDtypeStruct((N, hid), jnp.float32),
        ],
    )(Adj_param, x, W1, b1.reshape(1, hid))

    deg = 0.5 * (rowsum[:, 0] + jnp.sum(colpart, axis=(0, 1)))
    dinv = 1.0 / (jnp.sqrt(deg) + EOS)
    dc = dinv[:, None]
    dr = dinv[None, :]

    adjn, y1 = pl.pallas_call(
        _main_kernel,
        grid=(nb, nb),
        in_specs=[
            pl.BlockSpec((BM, BN), lambda i, j: (i, j)),
            pl.BlockSpec((BN, BM), lambda i, j: (j, i)),
            pl.BlockSpec((BN, hid), lambda i, j: (j, 0)),
            pl.BlockSpec((BM, 1), lambda i, j: (i, 0)),
            pl.BlockSpec((1, BN), lambda i, j: (0, j)),
        ],
        out_specs=[
            pl.BlockSpec((BM, BN), lambda i, j: (i, j)),
            pl.BlockSpec((BM, hid), lambda i, j: (i, 0)),
        ],
        out_shape=[
            jax.ShapeDtypeStruct((N, N), jnp.float32),
            jax.ShapeDtypeStruct((N, hid), jnp.float32),
        ],
    )(Adj_param, Adj_param, h1, dc, dr)

    out = pl.pallas_call(
        _out_kernel,
        grid=(nb, nb),
        in_specs=[
            pl.BlockSpec((BM, BN), lambda i, j: (i, j)),
            pl.BlockSpec((BN, hid), lambda i, j: (j, 0)),
            pl.BlockSpec((hid, ncls), lambda i, j: (0, 0)),
            pl.BlockSpec((1, ncls), lambda i, j: (0, 0)),
        ],
        out_specs=pl.BlockSpec((BM, ncls), lambda i, j: (i, 0)),
        out_shape=jax.ShapeDtypeStruct((N, ncls), jnp.float32),
        scratch_shapes=[pltpu.VMEM((N, ncls), jnp.float32)],
    )(adjn, y1, W2, b2.reshape(1, ncls))

    return (out, adjn)
